# baseline (speedup 1.0000x reference)
import functools

import jax
import jax.numpy as jnp
from jax import lax
from jax.experimental import pallas as pl
from jax.experimental.pallas import tpu as pltpu
from jax.experimental.pallas import tpu_sc as plsc

_NC = 2
_NS = 16
_LN_EPS = 1e-5
_C = 128
_NBUF = 3


def _zero_slice(acc_sh, zbuf, zrows, r0, nrows):
    done = 0
    while done + zrows <= nrows:
        pltpu.sync_copy(zbuf, acc_sh.at[pl.ds(r0 + done, zrows)])
        done += zrows
    if done < nrows:
        pltpu.sync_copy(zbuf.at[pl.ds(0, nrows - done)],
                        acc_sh.at[pl.ds(r0 + done, nrows - done)])


def _make_deg_kernel(NP, EP):
    NW = _NC * _NS
    EPW = EP // NW
    NCHUNK = EPW // _C
    assert NCHUNK % _NBUF == 0
    WR = (NP // _NS) // 8 * 8
    TAIL0 = _NS * WR
    TAIL = NP - TAIL0
    mesh = plsc.VectorSubcoreMesh(core_axis_name="c", subcore_axis_name="s")

    @functools.partial(
        pl.kernel,
        out_type=jax.ShapeDtypeStruct((_NC, NP, 16), jnp.float32),
        mesh=mesh,
        scratch_types=[
            pltpu.VMEM_SHARED((NP, 16), jnp.float32),
            pltpu.VMEM((NCHUNK, _C), jnp.int32),
            pltpu.VMEM((_C, 16), jnp.float32),
            pltpu.VMEM((128, 16), jnp.float32),
        ] + [pltpu.SemaphoreType.DMA] * _NBUF,
    )
    def deg_kernel(dst_hbm, out_hbm, acc_sh, didx, ones_v, zbuf, *ssem):
        c = lax.axis_index("c")
        s = lax.axis_index("s")
        w = c * _NS + s

        def fill(i, _):
            zbuf[i, :] = jnp.zeros((16,), jnp.float32)
            ones_v[i, :] = jnp.ones((16,), jnp.float32)
            return 0

        lax.fori_loop(0, 128, fill, 0)

        r0 = s * WR
        _zero_slice(acc_sh, zbuf, 128, r0, WR)

        @pl.when(s == _NS - 1)
        def _ztail():
            _zero_slice(acc_sh, zbuf, 128, TAIL0, TAIL)

        pltpu.sync_copy(dst_hbm.at[w], didx)
        plsc.subcore_barrier()

        def outer(g, _):
            for b in range(_NBUF):
                j = g * _NBUF + b

                @pl.when(g > 0)
                def _drain():
                    pltpu.make_async_copy(
                        ones_v, acc_sh.at[didx.at[0]], ssem[b]).wait()

                pltpu.async_copy(ones_v, acc_sh.at[didx.at[j]], ssem[b],
                                 add=True)
            return 0

        lax.fori_loop(0, NCHUNK // _NBUF, outer, 0)
        for b in range(_NBUF):
            pltpu.make_async_copy(ones_v, acc_sh.at[didx.at[0]],
                                  ssem[b]).wait()
        plsc.subcore_barrier()
        pltpu.sync_copy(acc_sh.at[pl.ds(r0, WR)],
                        out_hbm.at[c, pl.ds(r0, WR)])

        @pl.when(s == _NS - 1)
        def _wtail():
            pltpu.sync_copy(acc_sh.at[pl.ds(TAIL0, TAIL)],
                            out_hbm.at[c, pl.ds(TAIL0, TAIL)])

    return deg_kernel


def _make_agg_kernel(NP, EP, D):
    NW = _NC * _NS
    EPW = EP // NW
    NCHUNK = EPW // _C
    NOUTER = NCHUNK // _NBUF
    assert NCHUNK % _NBUF == 0
    WR = (NP // _NS) // 8 * 8
    TAIL0 = _NS * WR
    TAIL = NP - TAIL0
    mesh = plsc.VectorSubcoreMesh(core_axis_name="c", subcore_axis_name="s")

    @functools.partial(
        pl.kernel,
        out_type=jax.ShapeDtypeStruct((_NC, NP, D), jnp.float32),
        mesh=mesh,
        scratch_types=[
            pltpu.VMEM_SHARED((NP, D), jnp.float32),
            pltpu.VMEM((_NBUF, _C), jnp.int32),
            pltpu.VMEM((_NBUF, _C), jnp.int32),
            pltpu.VMEM((_NBUF, _C, D), jnp.float32),
        ] + [pltpu.SemaphoreType.DMA] * (3 * _NBUF),
    )
    def agg_kernel(u_hbm, src_hbm, dst_hbm, out_hbm,
                   acc_sh, sring, dring, rows, *sems):
        isem = sems[:_NBUF]
        dsem = sems[_NBUF:2 * _NBUF]
        gsem = sems[2 * _NBUF:]
        c = lax.axis_index("c")
        s = lax.axis_index("s")
        w = c * _NS + s

        def _idx_load(chunk, b):
            pltpu.async_copy(src_hbm.at[w, chunk], sring.at[b], isem[b])
            pltpu.async_copy(dst_hbm.at[w, chunk], dring.at[b], dsem[b])

        def _idx_wait(b, sem):
            pltpu.make_async_copy(src_hbm.at[w, 0], sring.at[b],
                                  sem).wait()

        def _gather_issue(b):
            pltpu.async_copy(u_hbm.at[sring.at[b]], rows.at[b], gsem[b])

        def _gather_wait(b):
            pltpu.make_async_copy(u_hbm.at[sring.at[b]], rows.at[b],
                                  gsem[b]).wait()

        for b in range(_NBUF):
            _idx_load(b, b)

        def fill(i, _):
            for k in range(D // 16):
                rows[0, i, pl.ds(16 * k, 16)] = jnp.zeros((16,), jnp.float32)
            return 0

        lax.fori_loop(0, _C, fill, 0)
        zbuf = rows.at[0]

        r0 = s * WR
        _zero_slice(acc_sh, zbuf, _C, r0, WR)

        @pl.when(s == _NS - 1)
        def _ztail():
            _zero_slice(acc_sh, zbuf, _C, TAIL0, TAIL)

        plsc.subcore_barrier()

        for j0 in range(2):
            _idx_wait(j0, isem[j0])
            _gather_issue(j0)

        def outer(g, _):
            for b in range(_NBUF):
                j = g * _NBUF + b
                b2 = (b + 2) % _NBUF
                _gather_wait(b)
                pltpu.make_async_copy(dst_hbm.at[w, 0], dring.at[b],
                                      dsem[b]).wait()
                pltpu.sync_copy(rows.at[b], acc_sh.at[dring.at[b]], add=True)

                @pl.when(g < NOUTER - 1)
                def _nidx():
                    _idx_load(j + _NBUF, b)

                @pl.when(j + 2 < NCHUNK)
                def _ngather():
                    _idx_wait(b2, isem[b2])
                    _gather_issue(b2)
            return 0

        lax.fori_loop(0, NOUTER, outer, 0)
        plsc.subcore_barrier()
        pltpu.sync_copy(acc_sh.at[pl.ds(r0, WR)],
                        out_hbm.at[c, pl.ds(r0, WR)])

        @pl.when(s == _NS - 1)
        def _wtail():
            pltpu.sync_copy(acc_sh.at[pl.ds(TAIL0, TAIL)],
                            out_hbm.at[c, pl.ds(TAIL0, TAIL)])

    return agg_kernel


def _dinv_from_parts(degp):
    deg = degp[0, :, 0] + degp[1, :, 0] + 1.0
    return lax.rsqrt(deg)[:, None]


def _pre_tc(x, W, degp, R=1000):
    N, D = x.shape

    def body(x_ref, w_ref, degp_ref, o_ref):
        dinv = _dinv_from_parts(degp_ref[...])
        o_ref[...] = jnp.dot(x_ref[...], w_ref[...],
                             preferred_element_type=jnp.float32) * dinv

    return pl.pallas_call(
        body,
        grid=(N // R,),
        in_specs=[
            pl.BlockSpec((R, D), lambda i: (i, 0)),
            pl.BlockSpec((D, D), lambda i: (0, 0)),
            pl.BlockSpec((2, R, 16), lambda i: (0, i, 0)),
        ],
        out_specs=pl.BlockSpec((R, D), lambda i: (i, 0)),
        out_shape=jax.ShapeDtypeStruct((N, D), jnp.float32),
    )(x, W, degp)


def _layer_post(t, g_v, be_v):
    mu = jnp.mean(t, axis=-1, keepdims=True)
    tcen = t - mu
    var = jnp.mean(tcen * tcen, axis=-1, keepdims=True)
    y = tcen * lax.rsqrt(var + _LN_EPS) * g_v + be_v
    return jnp.maximum(y, 0.0)


def _mid_tc(aggp, u, degp, b, g, be, Wn, R=1000):
    N, D = u.shape

    def body(aggp_ref, u_ref, degp_ref, b_ref, g_ref, be_ref, w_ref, o_ref):
        dinv = _dinv_from_parts(degp_ref[...])
        a = aggp_ref[...]
        t = dinv * (a[0] + a[1] + u_ref[...]) + b_ref[...]
        h = _layer_post(t, g_ref[...], be_ref[...])
        o_ref[...] = jnp.dot(h, w_ref[...],
                             preferred_element_type=jnp.float32) * dinv

    return pl.pallas_call(
        body,
        grid=(N // R,),
        in_specs=[
            pl.BlockSpec((2, R, D), lambda i: (0, i, 0)),
            pl.BlockSpec((R, D), lambda i: (i, 0)),
            pl.BlockSpec((2, R, 16), lambda i: (0, i, 0)),
            pl.BlockSpec((D,), lambda i: (0,)),
            pl.BlockSpec((D,), lambda i: (0,)),
            pl.BlockSpec((D,), lambda i: (0,)),
            pl.BlockSpec((D, D), lambda i: (0, 0)),
        ],
        out_specs=pl.BlockSpec((R, D), lambda i: (i, 0)),
        out_shape=jax.ShapeDtypeStruct((N, D), jnp.float32),
    )(aggp, u, degp, b, g, be, Wn)


def _final_tc(aggp, u, degp, b, g, be, batch, graph_attr,
              fc1_W, fc1_b, fc2_W, fc2_b, R=1000):
    N, D = u.shape
    B, G = graph_attr.shape
    OUT = fc2_W.shape[1]
    nblk = N // R

    def body(aggp_ref, u_ref, degp_ref, b_ref, g_ref, be_ref, batch_ref,
             ga_ref, fc1w_ref, fc1b_ref, fc2w_ref, fc2b_ref, o_ref,
             sum_ref, cnt_ref):
        i = pl.program_id(0)

        @pl.when(i == 0)
        def _init():
            sum_ref[...] = jnp.zeros_like(sum_ref)
            cnt_ref[...] = jnp.zeros_like(cnt_ref)

        dinv = _dinv_from_parts(degp_ref[...])
        a = aggp_ref[...]
        t = dinv * (a[0] + a[1] + u_ref[...]) + b_ref[...]
        h = _layer_post(t, g_ref[...], be_ref[...])

        batch_blk = batch_ref[...].reshape(R)
        onehot = (batch_blk[:, None] ==
                  lax.broadcasted_iota(jnp.int32, (R, B), 1)
                  ).astype(jnp.float32)
        sum_ref[...] += lax.dot_general(onehot, h, (((0,), (0,)), ((), ())),
                                        preferred_element_type=jnp.float32)
        cnt_ref[...] += jnp.sum(onehot, axis=0)[None, :]

        @pl.when(i == nblk - 1)
        def _fin():
            pooled = sum_ref[...] / jnp.maximum(cnt_ref[0, :], 1.0)[:, None]
            fc1w = fc1w_ref[...]
            z = (jnp.dot(pooled, fc1w[:D], preferred_element_type=jnp.float32)
                 + jnp.dot(ga_ref[...], fc1w[D:],
                           preferred_element_type=jnp.float32)
                 + fc1b_ref[...])
            z = jnp.maximum(z, 0.0)
            o_ref[...] = (jnp.dot(z, fc2w_ref[...],
                                  preferred_element_type=jnp.float32)
                          + fc2b_ref[...])

    return pl.pallas_call(
        body,
        grid=(nblk,),
        in_specs=[
            pl.BlockSpec((2, R, D), lambda i: (0, i, 0)),
            pl.BlockSpec((R, D), lambda i: (i, 0)),
            pl.BlockSpec((2, R, 16), lambda i: (0, i, 0)),
            pl.BlockSpec((D,), lambda i: (0,)),
            pl.BlockSpec((D,), lambda i: (0,)),
            pl.BlockSpec((D,), lambda i: (0,)),
            pl.BlockSpec((1, 1, R), lambda i: (i, 0, 0)),
            pl.BlockSpec((B, G), lambda i: (0, 0)),
            pl.BlockSpec(fc1_W.shape, lambda i: (0, 0)),
            pl.BlockSpec((D,), lambda i: (0,)),
            pl.BlockSpec((D, OUT), lambda i: (0, 0)),
            pl.BlockSpec((OUT,), lambda i: (0,)),
        ],
        out_specs=pl.BlockSpec((B, OUT), lambda i: (0, 0)),
        out_shape=jax.ShapeDtypeStruct((B, OUT), jnp.float32),
        scratch_shapes=[
            pltpu.VMEM((B, D), jnp.float32),
            pltpu.VMEM((1, B), jnp.float32),
        ],
    )(aggp, u, degp, b, g, be, batch.reshape(nblk, 1, R), graph_attr,
      fc1_W, fc1_b, fc2_W, fc2_b)


def kernel(x, edge_index, batch, graph_attr, W1, b1, W2, b2, W3, b3,
           g1, be1, g2, be2, g3, be3, fc1_W, fc1_b, fc2_W, fc2_b):
    N, D = x.shape
    E = edge_index.shape[1]
    NW = _NC * _NS
    NP = ((N + 16 + 7) // 8) * 8
    EQ = NW * _C * _NBUF
    EP = ((E + EQ - 1) // EQ) * EQ
    npad = EP - E

    src_p = jnp.concatenate(
        [edge_index[0], jnp.zeros((npad,), jnp.int32)])
    dst_p = jnp.concatenate(
        [edge_index[1],
         N + (jnp.arange(npad, dtype=jnp.int32) % 16)])
    src32 = src_p.reshape(NW, EP // (NW * _C), _C)
    dst32 = dst_p.reshape(NW, EP // (NW * _C), _C)

    deg_k = _make_deg_kernel(NP, EP)
    agg_k = _make_agg_kernel(NP, EP, D)

    degp = deg_k(dst32)
    u1 = _pre_tc(x, W1, degp)
    a1 = agg_k(u1, src32, dst32)
    u2 = _mid_tc(a1, u1, degp, b1, g1, be1, W2)
    a2 = agg_k(u2, src32, dst32)
    u3 = _mid_tc(a2, u2, degp, b2, g2, be2, W3)
    a3 = agg_k(u3, src32, dst32)
    return _final_tc(a3, u3, degp, b3, g3, be3, batch, graph_attr,
                     fc1_W, fc1_b, fc2_W, fc2_b)
